# Initial kernel scaffold; baseline (speedup 1.0000x reference)
import jax
import jax.numpy as jnp
from jax.experimental import pallas as pl


def kernel(batch, word_embeddings):
    raise NotImplementedError("write your pallas kernel here")


# speedup vs baseline: 4.8086x; 4.8086x over previous
import functools

import jax
import jax.numpy as jnp
from jax import lax
from jax.experimental import pallas as pl
from jax.experimental.pallas import tpu as pltpu
from jax.experimental.pallas import tpu_sc as plsc

EMBED_DIM = 32
NUM_CORES = 2
NUM_SUBCORES = 16
NW = NUM_CORES * NUM_SUBCORES
CHUNK = 1024


def _make_gather(b_flat):
    assert b_flat % (NW * CHUNK) == 0
    b_per_w = b_flat // NW
    n_chunks = b_per_w // CHUNK

    mesh = plsc.VectorSubcoreMesh(
        core_axis_name="c", subcore_axis_name="s",
        num_cores=NUM_CORES, num_subcores=NUM_SUBCORES)

    @functools.partial(
        pl.kernel,
        out_type=jax.ShapeDtypeStruct((b_flat, EMBED_DIM), jnp.float32),
        mesh=mesh,
        scratch_types=[
            pltpu.VMEM((CHUNK,), jnp.int32),
            pltpu.VMEM((CHUNK, EMBED_DIM), jnp.float32),
            pltpu.SemaphoreType.DMA,
        ],
        compiler_params=pltpu.CompilerParams(use_tc_tiling_on_sc=False),
    )
    def gather_kernel(idx_hbm, table_hbm, out_hbm, idx_v, rows_v, sem):
        wid = lax.axis_index("s") * NUM_CORES + lax.axis_index("c")
        base = wid * b_per_w

        def body(i, _):
            off = base + i * CHUNK
            pltpu.sync_copy(idx_hbm.at[pl.ds(off, CHUNK)], idx_v)
            pltpu.async_copy(table_hbm.at[idx_v], rows_v, sem).wait()
            pltpu.sync_copy(rows_v, out_hbm.at[pl.ds(off, CHUNK)])
            return 0

        lax.fori_loop(0, n_chunks, body, 0)

    return gather_kernel


def kernel(batch, word_embeddings):
    b, h = batch.shape
    flat = batch.reshape(b * h)
    out = _make_gather(b * h)(flat, word_embeddings)
    return out.reshape(b, h, EMBED_DIM)


# speedup vs baseline: 5.0471x; 1.0496x over previous
import functools

import jax
import jax.numpy as jnp
from jax import lax
from jax.experimental import pallas as pl
from jax.experimental.pallas import tpu as pltpu
from jax.experimental.pallas import tpu_sc as plsc

EMBED_DIM = 32
NUM_CORES = 2
NUM_SUBCORES = 16
NW = NUM_CORES * NUM_SUBCORES
CHUNK = 1024
NBUF = 2


def _make_gather(b_flat):
    assert b_flat % (NW * CHUNK) == 0
    b_per_w = b_flat // NW
    n = b_per_w // CHUNK
    assert n >= 4 and n % 2 == 0

    mesh = plsc.VectorSubcoreMesh(
        core_axis_name="c", subcore_axis_name="s",
        num_cores=NUM_CORES, num_subcores=NUM_SUBCORES)

    @functools.partial(
        pl.kernel,
        out_type=jax.ShapeDtypeStruct((b_flat, EMBED_DIM), jnp.float32),
        mesh=mesh,
        scratch_types=[
            pltpu.VMEM((CHUNK,), jnp.int32),
            pltpu.VMEM((CHUNK,), jnp.int32),
            pltpu.VMEM((CHUNK, EMBED_DIM), jnp.float32),
            pltpu.VMEM((CHUNK, EMBED_DIM), jnp.float32),
            pltpu.SemaphoreType.DMA,
            pltpu.SemaphoreType.DMA,
            pltpu.SemaphoreType.DMA,
            pltpu.SemaphoreType.DMA,
            pltpu.SemaphoreType.DMA,
            pltpu.SemaphoreType.DMA,
        ],
        compiler_params=pltpu.CompilerParams(use_tc_tiling_on_sc=False),
    )
    def gather_kernel(idx_hbm, table_hbm, out_hbm,
                      idx_v0, idx_v1, rows_v0, rows_v1,
                      is0, is1, gs0, gs1, os0, os1):
        idx_v = [idx_v0, idx_v1]
        rows_v = [rows_v0, rows_v1]
        idx_sem = [is0, is1]
        gat_sem = [gs0, gs1]
        out_sem = [os0, os1]

        wid = lax.axis_index("s") * NUM_CORES + lax.axis_index("c")
        base = wid * b_per_w

        def idx_copy(i, b):
            pltpu.async_copy(
                idx_hbm.at[pl.ds(base + i * CHUNK, CHUNK)], idx_v[b],
                idx_sem[b])

        def wait_idx(b):
            pltpu.make_async_copy(
                idx_hbm.at[pl.ds(0, CHUNK)], idx_v[b], idx_sem[b]).wait()

        def gather_start(b):
            pltpu.async_copy(table_hbm.at[idx_v[b]], rows_v[b], gat_sem[b])

        def wait_gat(b):
            pltpu.make_async_copy(
                table_hbm.at[idx_v[b]], rows_v[b], gat_sem[b]).wait()

        def out_copy(i, b):
            pltpu.async_copy(
                rows_v[b], out_hbm.at[pl.ds(base + i * CHUNK, CHUNK)],
                out_sem[b])

        def wait_out(b):
            pltpu.make_async_copy(
                rows_v[b], out_hbm.at[pl.ds(0, CHUNK)], out_sem[b]).wait()

        def step_a(i, b, check_out):
            wait_idx(b)
            if check_out:
                wait_out(b)
            gather_start(b)

        def step_b(j, pb, prefetch):
            wait_gat(pb)
            out_copy(j, pb)
            if prefetch:
                idx_copy(j + NBUF, pb)

        idx_copy(0, 0)
        idx_copy(1, 1)
        step_a(0, 0, False)
        step_a(1, 1, False)
        step_b(0, 0, True)

        def group(g, carry):
            i0 = 2 + 2 * g
            step_a(i0, 0, True)
            step_b(i0 - 1, 1, True)
            step_a(i0 + 1, 1, True)
            step_b(i0, 0, True)
            return carry

        lax.fori_loop(0, (n - 4) // 2, group, 0)

        step_a(n - 2, 0, True)
        step_b(n - 3, 1, True)
        step_a(n - 1, 1, True)
        step_b(n - 2, 0, False)
        step_b(n - 1, 1, False)
        wait_out(0)
        wait_out(1)

    return gather_kernel


def kernel(batch, word_embeddings):
    b, h = batch.shape
    flat = batch.reshape(b * h)
    out = _make_gather(b * h)(flat, word_embeddings)
    return out.reshape(b, h, EMBED_DIM)


# speedup vs baseline: 5.0508x; 1.0007x over previous
import functools

import jax
import jax.numpy as jnp
from jax import lax
from jax.experimental import pallas as pl
from jax.experimental.pallas import tpu as pltpu
from jax.experimental.pallas import tpu_sc as plsc

EMBED_DIM = 32
NUM_CORES = 2
NUM_SUBCORES = 16
NW = NUM_CORES * NUM_SUBCORES
CHUNK = 512
NBUF = 4
DRAIN = NBUF - 1


def _make_gather(b_flat):
    assert b_flat % (NW * CHUNK) == 0
    b_per_w = b_flat // NW
    n = b_per_w // CHUNK
    assert n >= 2 * NBUF

    mesh = plsc.VectorSubcoreMesh(
        core_axis_name="c", subcore_axis_name="s",
        num_cores=NUM_CORES, num_subcores=NUM_SUBCORES)

    scratch = (
        [pltpu.VMEM((CHUNK,), jnp.int32) for _ in range(NBUF)]
        + [pltpu.VMEM((CHUNK, EMBED_DIM), jnp.float32) for _ in range(NBUF)]
        + [pltpu.SemaphoreType.DMA for _ in range(3 * NBUF)]
    )

    @functools.partial(
        pl.kernel,
        out_type=jax.ShapeDtypeStruct((b_flat, EMBED_DIM), jnp.float32),
        mesh=mesh,
        scratch_types=scratch,
        compiler_params=pltpu.CompilerParams(use_tc_tiling_on_sc=False),
    )
    def gather_kernel(idx_hbm, table_hbm, out_hbm, *refs):
        idx_v = refs[:NBUF]
        rows_v = refs[NBUF:2 * NBUF]
        idx_sem = refs[2 * NBUF:3 * NBUF]
        gat_sem = refs[3 * NBUF:4 * NBUF]
        out_sem = refs[4 * NBUF:5 * NBUF]

        wid = lax.axis_index("s") * NUM_CORES + lax.axis_index("c")
        base = wid * b_per_w

        def idx_copy(i, b):
            pltpu.async_copy(
                idx_hbm.at[pl.ds(base + i * CHUNK, CHUNK)], idx_v[b],
                idx_sem[b])

        def wait_idx(b):
            pltpu.make_async_copy(
                idx_hbm.at[pl.ds(0, CHUNK)], idx_v[b], idx_sem[b]).wait()

        def gather_start(b):
            pltpu.async_copy(table_hbm.at[idx_v[b]], rows_v[b], gat_sem[b])

        def wait_gat(b):
            pltpu.make_async_copy(
                table_hbm.at[idx_v[b]], rows_v[b], gat_sem[b]).wait()

        def out_copy(i, b):
            pltpu.async_copy(
                rows_v[b], out_hbm.at[pl.ds(base + i * CHUNK, CHUNK)],
                out_sem[b])

        def wait_out(b):
            pltpu.make_async_copy(
                rows_v[b], out_hbm.at[pl.ds(0, CHUNK)], out_sem[b]).wait()

        def step_a(i, b, check_out):
            wait_idx(b)
            if check_out:
                wait_out(b)
            gather_start(b)

        def step_b(j, pb, prefetch):
            wait_gat(pb)
            out_copy(j, pb)
            if prefetch:
                idx_copy(j + NBUF, pb)

        for b in range(NBUF):
            idx_copy(b, b)
        for i in range(DRAIN):
            step_a(i, i % NBUF, False)

        for i in range(DRAIN, DRAIN + NBUF):
            step_a(i, i % NBUF, i >= NBUF)
            step_b(i - DRAIN, (i + 1) % NBUF, True)

        start = DRAIN + NBUF
        n_mid = (n - start) // NBUF

        def group(g, carry):
            i0 = start + g * NBUF
            for t in range(NBUF):
                b = (start + t) % NBUF
                step_a(i0 + t, b, True)
                step_b(i0 + t - DRAIN, (b + 1) % NBUF, True)
            return carry

        lax.fori_loop(0, n_mid, group, 0)

        for i in range(start + n_mid * NBUF, n):
            step_a(i, i % NBUF, True)
            step_b(i - DRAIN, (i + 1) % NBUF, i + 1 < n)

        for j in range(n - DRAIN, n):
            step_b(j, j % NBUF, False)
        for b in range(NBUF):
            wait_out(b)

    return gather_kernel


def kernel(batch, word_embeddings):
    b, h = batch.shape
    flat = batch.reshape(b * h)
    out = _make_gather(b * h)(flat, word_embeddings)
    return out.reshape(b, h, EMBED_DIM)
